# Initial kernel scaffold; baseline (speedup 1.0000x reference)
import jax
import jax.numpy as jnp
from jax.experimental import pallas as pl


def kernel(X, codebook, iters):
    raise NotImplementedError("write your pallas kernel here")


# speedup vs baseline: 1.2001x; 1.2001x over previous
import jax
import jax.numpy as jnp
from jax import lax
from jax.experimental import pallas as pl
from jax.experimental.pallas import tpu as pltpu

_T = 1024


def _kmeans_body(iters_ref, x_ref, cb0_ref, out_ref):
    n, d = x_ref.shape
    kk = cb0_ref.shape[0]
    nt = n // _T
    out_ref[:] = cb0_ref[:]

    iota0 = lax.broadcasted_iota(jnp.int32, (kk, _T), 0)

    def outer(_, carry):
        cb = out_ref[:]
        c2 = jnp.sum(cb * cb, axis=1, keepdims=True)

        cb16 = cb.astype(jnp.bfloat16)

        def tile(j, acc):
            sums, counts = acc
            x = x_ref[pl.ds(j * _T, _T), :]
            g = lax.dot_general(cb16, x.astype(jnp.bfloat16),
                                (((1,), (1,)), ((), ())),
                                preferred_element_type=jnp.float32)
            dt = c2 - 2.0 * g
            m = jnp.min(dt, axis=0, keepdims=True)
            idx = jnp.min(jnp.where(dt == m, iota0, kk), axis=0, keepdims=True)
            oh = (iota0 == idx).astype(jnp.float32)
            sums = sums + lax.dot_general(oh, x, (((1,), (0,)), ((), ())),
                                          preferred_element_type=jnp.float32,
                                          precision=lax.Precision.HIGHEST)
            counts = counts + jnp.sum(oh, axis=1, keepdims=True)
            return sums, counts

        sums0 = jnp.zeros((kk, d), jnp.float32)
        counts0 = jnp.zeros((kk, 1), jnp.float32)
        sums, counts = lax.fori_loop(0, nt, tile, (sums0, counts0))
        mean = sums / jnp.maximum(counts, 1.0)
        out_ref[:] = jnp.where(counts > 0.0, mean, cb)
        return carry

    lax.fori_loop(0, iters_ref[0], outer, 0)


def kernel(X, codebook, iters):
    n, d = X.shape
    kk = codebook.shape[0]
    idx = jax.random.permutation(jax.random.key(42), n)[:kk]
    cb0 = X[idx]
    it = jnp.asarray(iters, jnp.int32).reshape(1)
    return pl.pallas_call(
        _kmeans_body,
        out_shape=jax.ShapeDtypeStruct((kk, d), X.dtype),
        in_specs=[
            pl.BlockSpec(memory_space=pltpu.SMEM),
            pl.BlockSpec(memory_space=pltpu.VMEM),
            pl.BlockSpec(memory_space=pltpu.VMEM),
        ],
        out_specs=pl.BlockSpec(memory_space=pltpu.VMEM),
    )(it, X, cb0)


# speedup vs baseline: 2.6216x; 2.1844x over previous
import jax
import jax.numpy as jnp
from jax import lax
from jax.experimental import pallas as pl
from jax.experimental.pallas import tpu as pltpu

_T = 1024


def _kmeans_body(iters_ref, x_ref, cb0_ref, out_ref):
    n, d = x_ref.shape
    kk = cb0_ref.shape[0]
    nt = n // _T
    out_ref[:] = cb0_ref[:]

    iota0 = lax.broadcasted_iota(jnp.int32, (kk, _T), 0)

    def outer(_, carry):
        cb = out_ref[:]
        c2 = jnp.sum(cb * cb, axis=1, keepdims=True)

        cb16 = cb.astype(jnp.bfloat16)

        def tile(j, acc):
            sums, counts = acc
            x = x_ref[pl.ds(j * _T, _T), :]
            g = lax.dot_general(cb16, x.astype(jnp.bfloat16),
                                (((1,), (1,)), ((), ())),
                                preferred_element_type=jnp.float32)
            dt = c2 - 2.0 * g
            m = jnp.min(dt, axis=0, keepdims=True)
            idx = jnp.min(jnp.where(dt == m, iota0, kk), axis=0, keepdims=True)
            oh = (iota0 == idx).astype(jnp.float32)
            sums = sums + lax.dot_general(oh, x, (((1,), (0,)), ((), ())),
                                          preferred_element_type=jnp.float32)
            counts = counts + jnp.sum(oh, axis=1, keepdims=True)
            return sums, counts

        sums0 = jnp.zeros((kk, d), jnp.float32)
        counts0 = jnp.zeros((kk, 1), jnp.float32)
        sums, counts = lax.fori_loop(0, nt, tile, (sums0, counts0))
        mean = sums / jnp.maximum(counts, 1.0)
        out_ref[:] = jnp.where(counts > 0.0, mean, cb)
        return carry

    lax.fori_loop(0, iters_ref[0], outer, 0)


def kernel(X, codebook, iters):
    n, d = X.shape
    kk = codebook.shape[0]
    idx = jax.random.permutation(jax.random.key(42), n)[:kk]
    cb0 = X[idx]
    it = jnp.asarray(iters, jnp.int32).reshape(1)
    return pl.pallas_call(
        _kmeans_body,
        out_shape=jax.ShapeDtypeStruct((kk, d), X.dtype),
        in_specs=[
            pl.BlockSpec(memory_space=pltpu.SMEM),
            pl.BlockSpec(memory_space=pltpu.VMEM),
            pl.BlockSpec(memory_space=pltpu.VMEM),
        ],
        out_specs=pl.BlockSpec(memory_space=pltpu.VMEM),
    )(it, X, cb0)
